# baseline (speedup 1.0000x reference)
import functools

import jax
import jax.numpy as jnp
import numpy as np
from jax import lax
from jax.experimental import pallas as pl
from jax.experimental.pallas import tpu as pltpu
from jax.experimental.pallas import tpu_sc as plsc

NN = 524288
FEAT = 32
H, W = 128, 64
NROWS = H * W

NC, NS, L = 2, 16, 16
NWORK = NC * NS
CH = 512
GB = 128
NB = CH // GB
PER_W = NN // NWORK
CHUNKS = PER_W // CH

SCALE = float(np.float32(2.0) / (np.float32(-1.3) - np.float32(1.3)))
TS = float(np.float32(1.0 / 128.0))


def _indices(pv, tv):
    x = (tv + 1.0) * 0.5 * float(W - 1)
    ix = jnp.minimum(jnp.maximum((x + 0.5).astype(jnp.int32), 0), W - 1)
    p = (pv - 1.3) * SCALE - 1.0
    y = (p + 1.0) * 0.5 * float(H - 1)
    iy = jnp.minimum(jnp.maximum((y + 0.5).astype(jnp.int32), 0), H - 1)
    return iy * W + ix


def _sc_sample(ptsT, tab0, tab1, tab2):
    mesh = plsc.VectorSubcoreMesh(core_axis_name="c", subcore_axis_name="s")
    out_t = jax.ShapeDtypeStruct((NN, FEAT), jnp.float32)
    cbuf_t = pltpu.VMEM((4, CH), jnp.float32)
    ibuf_t = pltpu.VMEM((NB, GB), jnp.int32)
    rbuf_t = pltpu.VMEM((CH, FEAT), jnp.float32)

    @functools.partial(
        pl.kernel,
        out_type=[out_t, out_t],
        mesh=mesh,
        compiler_params=pltpu.CompilerParams(
            use_tc_tiling_on_sc=False, needs_layout_passes=False),
        scratch_types=[
            [cbuf_t, cbuf_t],
            [[ibuf_t] * 3, [ibuf_t] * 3],
            [[rbuf_t] * 3, [rbuf_t] * 3],
            pltpu.VMEM((4, L), jnp.float32),
            pltpu.VMEM((3, L), jnp.int32),
            pltpu.VMEM((3, L, FEAT), jnp.float32),
            pltpu.VMEM((FEAT,), jnp.float32),
            pltpu.VMEM((FEAT,), jnp.float32),
            [pltpu.SemaphoreType.DMA] * 2,
            [pltpu.SemaphoreType.DMA] * 2,
        ],
    )
    def body(ptsT_h, t0_h, t1_h, t2_h, outA, outB,
             cbufs, ibufs, rbufs, pbuf, fibuf, frbuf, fabuf, fbbuf,
             sem_g, sem_o):
        wid = lax.axis_index("s") * NC + lax.axis_index("c")
        wbase = wid * PER_W
        tabs = (t0_h, t1_h, t2_h)

        def load_coords(c, p):
            pltpu.sync_copy(ptsT_h.at[:, pl.ds(wbase + c * CH, CH)], cbufs[p])

        def gen_idx(p):
            cb = cbufs[p]
            ibs = ibufs[p]

            def idx_block(b, _):
                def idx_group(u, _):
                    o = b * GB + u * L
                    col = u * L
                    tv = cb[3, pl.ds(o, L)] * 2.0 - 1.0
                    ibs[0][b, pl.ds(col, L)] = _indices(cb[0, pl.ds(o, L)], tv)
                    ibs[1][b, pl.ds(col, L)] = _indices(cb[1, pl.ds(o, L)], tv)
                    ibs[2][b, pl.ds(col, L)] = _indices(cb[2, pl.ds(o, L)], tv)
                    return 0
                return lax.fori_loop(0, GB // L, idx_group, 0)
            lax.fori_loop(0, NB, idx_block, 0)

        def fire_gathers(p):
            for b in range(NB):
                dst = pl.ds(b * GB, GB)
                for t in range(3):
                    pltpu.async_copy(
                        tabs[t].at[ibufs[p][t].at[b]], rbufs[p][t].at[dst],
                        sem_g[p])

        def drain_gathers(p):
            for b in range(NB):
                dst = pl.ds(b * GB, GB)
                for t in range(3):
                    pltpu.make_async_copy(
                        tabs[t].at[ibufs[p][t].at[b]], rbufs[p][t].at[dst],
                        sem_g[p]).wait()

        def multiply(p):
            r0, r1, r2 = rbufs[p]

            def mul_row(j, _):
                for h in (0, L):
                    s = pl.ds(h, L)
                    r0[j, s] = (r0[j, s] * r1[j, s]) * r2[j, s]
                return 0
            lax.fori_loop(0, CH, mul_row, 0)

        def fire_outs(c, p):
            base = wbase + c * CH
            pltpu.async_copy(rbufs[p][0], outA.at[pl.ds(base, CH)], sem_o[p])
            pltpu.async_copy(rbufs[p][0], outB.at[pl.ds(base, CH)], sem_o[p])

        def drain_outs(p):
            for _ in range(2):
                pltpu.make_async_copy(
                    rbufs[p][0], outA.at[pl.ds(wbase, CH)], sem_o[p]).wait()

        load_coords(0, 0)
        gen_idx(0)
        fire_gathers(0)

        def chunk_body(c, _):
            p = lax.rem(c, 2)

            @pl.when(p == 0)
            def _even():
                _steady(c, 0, 1)

            @pl.when(p == 1)
            def _odd():
                _steady(c, 1, 0)
            return 0

        def _steady(c, p, q):
            @pl.when(c < CHUNKS - 1)
            def _prep():
                load_coords(c + 1, q)
                gen_idx(q)

            @pl.when(c >= 2)
            def _do():
                drain_outs(q)
            drain_gathers(p)

            @pl.when(c < CHUNKS - 1)
            def _fire():
                fire_gathers(q)
            multiply(p)
            fire_outs(c, p)

        lax.fori_loop(0, CHUNKS, chunk_body, 0)
        drain_outs(0)
        drain_outs(1)

        @pl.when(wid == NWORK - 1)
        def _fixup():
            pltpu.sync_copy(ptsT_h.at[:, pl.ds(NN - L, L)], pbuf)
            tv = (pbuf[3, :] * 2.0 - 1.0) - TS
            x = (tv + 1.0) * 0.5 * float(W - 1)
            ix = jnp.minimum(jnp.maximum((x + 0.5).astype(jnp.int32), 0), W - 1)
            for c in range(3):
                p = ((pbuf[c, :] - 1.3) * SCALE - 1.0) - TS
                y = (p + 1.0) * 0.5 * float(H - 1)
                iy = jnp.minimum(jnp.maximum((y + 0.5).astype(jnp.int32), 0), H - 1)
                fibuf[c, :] = iy * W + ix
            for t in range(3):
                pltpu.async_copy(
                    tabs[t].at[fibuf.at[t]], frbuf.at[t], sem_g[0]).wait()
            zc = jnp.zeros((L,), jnp.int32)
            pxl = plsc.load_gather(pbuf, [zc, jnp.full((L,), L - 1, jnp.int32)])
            cv = ((pxl - 1.3) * SCALE - 1.0) + TS > 1.0
            ulast = rbufs[(CHUNKS - 1) % 2][0]
            for h in (0, L):
                s = pl.ds(h, L)
                sh = (frbuf[0, L - 1, s] * frbuf[1, L - 1, s]) * frbuf[2, L - 1, s]
                u = ulast[CH - 1, s]
                fabuf[s] = jnp.where(cv, sh, u)
                fbbuf[s] = jnp.where(cv, u, sh)
            pltpu.sync_copy(fabuf, outA.at[NN - 1])
            pltpu.sync_copy(fbbuf, outB.at[NN - 1])

    return body(ptsT, tab0, tab1, tab2)


def kernel(pts, time, plane0, plane1, plane2):
    ptsT = jnp.concatenate([pts, time], axis=1).T
    eye = jnp.eye(FEAT, dtype=jnp.float32)
    tab0 = jnp.einsum("fhw,fg->hwg", plane0, eye).reshape(NROWS, FEAT)
    tab1 = jnp.einsum("fhw,fg->hwg", plane1, eye).reshape(NROWS, FEAT)
    tab2 = jnp.einsum("fhw,fg->hwg", plane2, eye).reshape(NROWS, FEAT)
    feature_a, feature_b = _sc_sample(ptsT, tab0, tab1, tab2)
    return (feature_a, feature_b)


# speedup vs baseline: 1.0120x; 1.0120x over previous
import functools

import jax
import jax.numpy as jnp
import numpy as np
from jax import lax
from jax.experimental import pallas as pl
from jax.experimental.pallas import tpu as pltpu
from jax.experimental.pallas import tpu_sc as plsc

NN = 524288
FEAT = 32
H, W = 128, 64
NROWS = H * W

NC, NS, L = 2, 16, 16
NWORK = NC * NS
CH = 512
GB = 128
NB = CH // GB
PER_W = NN // NWORK
CHUNKS = PER_W // CH

SCALE = float(np.float32(2.0) / (np.float32(-1.3) - np.float32(1.3)))
TS = float(np.float32(1.0 / 128.0))


def _indices(pv, tv):
    x = (tv + 1.0) * 0.5 * float(W - 1)
    ix = jnp.minimum(jnp.maximum((x + 0.5).astype(jnp.int32), 0), W - 1)
    p = (pv - 1.3) * SCALE - 1.0
    y = (p + 1.0) * 0.5 * float(H - 1)
    iy = jnp.minimum(jnp.maximum((y + 0.5).astype(jnp.int32), 0), H - 1)
    return iy * W + ix


def _sc_sample(ptsT, tab0, tab1, tab2):
    mesh = plsc.VectorSubcoreMesh(core_axis_name="c", subcore_axis_name="s")
    out_t = jax.ShapeDtypeStruct((NN, FEAT), jnp.float32)
    cbuf_t = pltpu.VMEM((4, CH), jnp.float32)
    ibuf_t = pltpu.VMEM((NB, GB), jnp.int32)
    rbuf_t = pltpu.VMEM((CH, FEAT), jnp.float32)

    @functools.partial(
        pl.kernel,
        out_type=[out_t, out_t],
        mesh=mesh,
        compiler_params=pltpu.CompilerParams(
            use_tc_tiling_on_sc=False, needs_layout_passes=False),
        scratch_types=[
            [cbuf_t, cbuf_t],
            [[ibuf_t] * 3, [ibuf_t] * 3],
            [[rbuf_t] * 3, [rbuf_t] * 3],
            pltpu.VMEM((4, L), jnp.float32),
            pltpu.VMEM((3, L), jnp.int32),
            pltpu.VMEM((3, L, FEAT), jnp.float32),
            pltpu.VMEM((FEAT,), jnp.float32),
            pltpu.VMEM((FEAT,), jnp.float32),
            [pltpu.SemaphoreType.DMA] * 2,
            [pltpu.SemaphoreType.DMA] * 2,
        ],
    )
    def body(ptsT_h, t0_h, t1_h, t2_h, outA, outB,
             cbufs, ibufs, rbufs, pbuf, fibuf, frbuf, fabuf, fbbuf,
             sem_g, sem_o):
        wid = lax.axis_index("s") * NC + lax.axis_index("c")
        wbase = wid * PER_W
        tabs = (t0_h, t1_h, t2_h)

        def load_coords(c, p):
            pltpu.sync_copy(ptsT_h.at[:, pl.ds(wbase + c * CH, CH)], cbufs[p])

        def gen_idx(p):
            cb = cbufs[p]
            ibs = ibufs[p]

            def idx_block(b, _):
                def idx_group(u, _):
                    o = b * GB + u * L
                    col = u * L
                    tv = cb[3, pl.ds(o, L)] * 2.0 - 1.0
                    ibs[0][b, pl.ds(col, L)] = _indices(cb[0, pl.ds(o, L)], tv)
                    ibs[1][b, pl.ds(col, L)] = _indices(cb[1, pl.ds(o, L)], tv)
                    ibs[2][b, pl.ds(col, L)] = _indices(cb[2, pl.ds(o, L)], tv)
                    return 0
                return lax.fori_loop(0, GB // L, idx_group, 0)
            lax.fori_loop(0, NB, idx_block, 0)

        def fire_gathers(p):
            for b in range(NB):
                dst = pl.ds(b * GB, GB)
                for t in range(3):
                    pltpu.async_copy(
                        tabs[t].at[ibufs[p][t].at[b]], rbufs[p][t].at[dst],
                        sem_g[p])

        def drain_gathers(p):
            for b in range(NB):
                dst = pl.ds(b * GB, GB)
                for t in range(3):
                    pltpu.make_async_copy(
                        tabs[t].at[ibufs[p][t].at[b]], rbufs[p][t].at[dst],
                        sem_g[p]).wait()

        def multiply(p):
            r0, r1, r2 = rbufs[p]

            def mul_row(j, _):
                for h in (0, L):
                    s = pl.ds(h, L)
                    r0[j, s] = (r0[j, s] * r1[j, s]) * r2[j, s]
                return 0
            lax.fori_loop(0, CH, mul_row, 0)

        def fire_outs(c, p):
            base = wbase + c * CH
            pltpu.async_copy(rbufs[p][0], outA.at[pl.ds(base, CH)], sem_o[p])
            pltpu.async_copy(rbufs[p][0], outB.at[pl.ds(base, CH)], sem_o[p])

        def drain_outs(p):
            for _ in range(2):
                pltpu.make_async_copy(
                    rbufs[p][0], outA.at[pl.ds(wbase, CH)], sem_o[p]).wait()

        load_coords(0, 0)
        gen_idx(0)
        fire_gathers(0)

        def chunk_body(c, _):
            p = lax.rem(c, 2)

            @pl.when(p == 0)
            def _even():
                _steady(c, 0, 1)

            @pl.when(p == 1)
            def _odd():
                _steady(c, 1, 0)
            return 0

        def _steady(c, p, q):
            @pl.when(c < CHUNKS - 1)
            def _prep():
                load_coords(c + 1, q)
                gen_idx(q)

            @pl.when(c >= 2)
            def _do():
                drain_outs(q)
            drain_gathers(p)

            @pl.when(c < CHUNKS - 1)
            def _fire():
                fire_gathers(q)
            multiply(p)
            fire_outs(c, p)

        lax.fori_loop(0, CHUNKS, chunk_body, 0)
        drain_outs(0)
        drain_outs(1)

        @pl.when(wid == NWORK - 1)
        def _fixup():
            pltpu.sync_copy(ptsT_h.at[:, pl.ds(NN - L, L)], pbuf)
            tv = (pbuf[3, :] * 2.0 - 1.0) - TS
            x = (tv + 1.0) * 0.5 * float(W - 1)
            ix = jnp.minimum(jnp.maximum((x + 0.5).astype(jnp.int32), 0), W - 1)
            for c in range(3):
                p = ((pbuf[c, :] - 1.3) * SCALE - 1.0) - TS
                y = (p + 1.0) * 0.5 * float(H - 1)
                iy = jnp.minimum(jnp.maximum((y + 0.5).astype(jnp.int32), 0), H - 1)
                fibuf[c, :] = iy * W + ix
            for t in range(3):
                pltpu.async_copy(
                    tabs[t].at[fibuf.at[t]], frbuf.at[t], sem_g[0]).wait()
            zc = jnp.zeros((L,), jnp.int32)
            pxl = plsc.load_gather(pbuf, [zc, jnp.full((L,), L - 1, jnp.int32)])
            cv = ((pxl - 1.3) * SCALE - 1.0) + TS > 1.0
            ulast = rbufs[(CHUNKS - 1) % 2][0]
            for h in (0, L):
                s = pl.ds(h, L)
                sh = (frbuf[0, L - 1, s] * frbuf[1, L - 1, s]) * frbuf[2, L - 1, s]
                u = ulast[CH - 1, s]
                fabuf[s] = jnp.where(cv, sh, u)
                fbbuf[s] = jnp.where(cv, u, sh)
            pltpu.sync_copy(fabuf, outA.at[NN - 1])
            pltpu.sync_copy(fbbuf, outB.at[NN - 1])

    return body(ptsT, tab0, tab1, tab2)


def kernel(pts, time, plane0, plane1, plane2):
    ptsT = jnp.concatenate([pts, time], axis=1).T
    tab0 = jnp.transpose(plane0, (1, 2, 0)).reshape(NROWS, FEAT)
    tab1 = jnp.transpose(plane1, (1, 2, 0)).reshape(NROWS, FEAT)
    tab2 = jnp.transpose(plane2, (1, 2, 0)).reshape(NROWS, FEAT)
    feature_a, feature_b = _sc_sample(ptsT, tab0, tab1, tab2)
    return (feature_a, feature_b)
